# baseline (speedup 1.0000x reference)
import functools

import jax
import jax.numpy as jnp
from jax import lax
from jax.experimental import pallas as pl
from jax.experimental.pallas import tpu as pltpu
from jax.experimental.pallas import tpu_sc as plsc

_W = 80
_NSUB = 16
_NCORE = 2
_F32 = jnp.float32


@functools.lru_cache(maxsize=None)
def _make_deg_kernel(Npad, E):
    nwin_w = E // _W // (_NCORE * _NSUB)
    r0 = Npad // _NSUB
    mesh = plsc.VectorSubcoreMesh(core_axis_name="c", subcore_axis_name="s")

    @functools.partial(
        pl.kernel,
        out_type=jax.ShapeDtypeStruct((_NCORE * Npad,), _F32),
        mesh=mesh,
        scratch_types=[
            pltpu.VMEM_SHARED((Npad,), _F32),
            pltpu.VMEM((nwin_w, _W), jnp.int32),
            pltpu.VMEM((_W,), _F32),
            pltpu.VMEM((r0,), _F32),
        ],
    )
    def deg_kernel(dst3d, out, acc, idxs, ones, zbuf):
        c = lax.axis_index("c")
        s = lax.axis_index("s")

        def fill_ones(i, carry):
            ones[pl.ds(i * 16, 16)] = jnp.ones((16,), _F32)
            return carry

        lax.fori_loop(0, _W // 16, fill_ones, 0)

        def fill_zeros(i, carry):
            zbuf[pl.ds(i * 16, 16)] = jnp.zeros((16,), _F32)
            return carry

        lax.fori_loop(0, r0 // 16, fill_zeros, 0)

        pltpu.sync_copy(zbuf, acc.at[pl.ds(s * r0, r0)])
        plsc.subcore_barrier()

        wid = c * _NSUB + s
        pltpu.sync_copy(dst3d.at[wid], idxs)

        def win(w, carry):
            pltpu.sync_copy(ones, acc.at[idxs.at[w]], add=True)
            return carry

        lax.fori_loop(0, nwin_w, win, 0)
        plsc.subcore_barrier()

        pltpu.sync_copy(acc.at[pl.ds(s * r0, r0)], zbuf)
        pltpu.sync_copy(zbuf, out.at[pl.ds(c * Npad + s * r0, r0)])

    return deg_kernel


_BW = 25


@functools.lru_cache(maxsize=None)
def _make_prop_kernel(N, Npad, E, H, nchunks, esplit=False):
    cpc = nchunks if esplit else nchunks // _NCORE
    nwin = E // _W // _NSUB
    nb = nwin // _BW
    nb_eff = nb // _NCORE if esplit else nb
    rows_s = Npad // _NSUB
    mesh = plsc.VectorSubcoreMesh(core_axis_name="c", subcore_axis_name="s")

    out_shape = ((_NCORE, nchunks, Npad, H) if esplit
                 else (nchunks, Npad, H))
    nring = 3
    nwtot = nb_eff * _BW
    @functools.partial(
        pl.kernel,
        out_type=jax.ShapeDtypeStruct(out_shape, _F32),
        mesh=mesh,
        scratch_types=(
            [pltpu.VMEM_SHARED((Npad, H), _F32)]
            + [pltpu.VMEM((_BW, _W), jnp.int32)] * 4
            + [pltpu.VMEM((_W, H), _F32)] * nring
            + [pltpu.SemaphoreType.DMA] * (2 * nring + 4)
        ),
    )
    def prop_kernel(vflat, src5d, dst4d, out, acc, *rest):
        sidxp = (rest[0], rest[2])
        didxp = (rest[1], rest[3])
        bufs = rest[4:4 + nring]
        gsems = rest[4 + nring:4 + 2 * nring]
        ssems = rest[4 + 2 * nring:4 + 3 * nring]
        isems = rest[4 + 3 * nring:]
        c = lax.axis_index("c")
        s = lax.axis_index("s")

        for ci in range(cpc):
            chunk = ci if esplit else c * cpc + ci

            def fill_zeros(i, carry):
                bufs[0][i // (H // 16), pl.ds((i % (H // 16)) * 16, 16)] = \
                    jnp.zeros((16,), _F32)
                return carry

            lax.fori_loop(0, _W * (H // 16), fill_zeros, 0)
            for kk in range(rows_s // _W):
                pltpu.sync_copy(bufs[0], acc.at[pl.ds(s * rows_s + kk * _W, _W)])
            plsc.subcore_barrier()

            icp = [None, None, None, None]

            def idx_load(wb):
                p = wb % 2
                wbi = c * nb_eff + wb if esplit else wb
                icp[2 * p] = pltpu.async_copy(
                    src5d.at[chunk, s, wbi], sidxp[p], isems[2 * p])
                icp[2 * p + 1] = pltpu.async_copy(
                    dst4d.at[s, wbi], didxp[p], isems[2 * p + 1])

            idx_load(0)
            if nb_eff > 1:
                idx_load(1)

            pf = nring - 1
            gcp = [None] * nring
            scp = [None] * nring

            def issue_gather(w):
                j = w % nring
                wb, row = w // _BW, w % _BW
                if row == 0:
                    icp[2 * (wb % 2)].wait()
                gcp[j] = pltpu.async_copy(
                    vflat.at[sidxp[wb % 2].at[row]], bufs[j], gsems[j])

            for w in range(min(pf, nwtot)):
                issue_gather(w)
            for w in range(nwtot):
                b = w % nring
                wb, row = w // _BW, w % _BW
                if row == 0:
                    icp[2 * (wb % 2) + 1].wait()
                nxt = w + pf
                if nxt < nwtot:
                    j = nxt % nring
                    if scp[j] is not None:
                        scp[j].wait()
                    issue_gather(nxt)
                gcp[b].wait()
                scp[b] = pltpu.async_copy(
                    bufs[b], acc.at[didxp[wb % 2].at[row]], ssems[b], add=True)
                if row == nring and 1 <= wb < nb_eff - 1:
                    idx_load(wb + 1)
            for j in range(nring):
                if scp[j] is not None:
                    scp[j].wait()
            plsc.subcore_barrier()

            for kk in range(rows_s // _W):
                ro = s * rows_s + kk * _W
                dst_ref = (out.at[c, chunk, pl.ds(ro, _W)] if esplit
                           else out.at[chunk, pl.ds(ro, _W)])
                pltpu.sync_copy(acc.at[pl.ds(ro, _W)], dst_ref)
            if ci + 1 < cpc:
                plsc.subcore_barrier()

    return prop_kernel


def _dinv_of(d0, d1):
    return lax.rsqrt(d0[...] + d1[...] + 1.0)


def _tc_scale(x, h, deg0, deg1, B, N, H, bn):
    def body(x_ref, h_ref, d0, d1, v_ref):
        dinv = _dinv_of(d0, d1)
        v_ref[0, 0] = x_ref[0] * dinv
        v_ref[0, 1] = h_ref[0] * dinv

    return pl.pallas_call(
        body,
        grid=(B, N // bn),
        in_specs=[
            pl.BlockSpec((1, bn, H), lambda b, i: (b, i, 0)),
            pl.BlockSpec((1, bn, H), lambda b, i: (b, i, 0)),
            pl.BlockSpec((bn, 1), lambda b, i: (i, 0)),
            pl.BlockSpec((bn, 1), lambda b, i: (i, 0)),
        ],
        out_specs=pl.BlockSpec((1, 2, bn, H), lambda b, i: (b, 0, i, 0)),
        out_shape=jax.ShapeDtypeStruct((B, 2, N, H), _F32),
    )(x, h, deg0, deg1)


def _tc_gate(acc1, v1, h, deg0, deg1, wzr, bz, br, B, N, H, bn):
    def body(acc_ref, v1_ref, h_ref, d0, d1, wzr_ref, bz_ref, br_ref,
             v2_ref, z_ref, px_ref):
        dinv = _dinv_of(d0, d1)
        px = (acc_ref[0, 0] + v1_ref[0, 0]) * dinv
        ph = (acc_ref[0, 1] + v1_ref[0, 1]) * dinv
        zr_pre = (jnp.dot(px, wzr_ref[0:H, :], preferred_element_type=_F32)
                  + jnp.dot(ph, wzr_ref[H:2 * H, :], preferred_element_type=_F32))
        z = jax.nn.sigmoid(zr_pre[:, 0:H] + bz_ref[...])
        r = jax.nn.sigmoid(zr_pre[:, H:2 * H] + br_ref[...])
        v2_ref[0] = (r * h_ref[0]) * dinv
        z_ref[0] = z
        px_ref[0] = px

    shp = jax.ShapeDtypeStruct((B, N, H), _F32)
    return pl.pallas_call(
        body,
        grid=(B, N // bn),
        in_specs=[
            pl.BlockSpec((1, 2, bn, H), lambda b, i: (b, 0, i, 0)),
            pl.BlockSpec((1, 2, bn, H), lambda b, i: (b, 0, i, 0)),
            pl.BlockSpec((1, bn, H), lambda b, i: (b, i, 0)),
            pl.BlockSpec((bn, 1), lambda b, i: (i, 0)),
            pl.BlockSpec((bn, 1), lambda b, i: (i, 0)),
            pl.BlockSpec((2 * H, 2 * H), lambda b, i: (0, 0)),
            pl.BlockSpec((1, H), lambda b, i: (0, 0)),
            pl.BlockSpec((1, H), lambda b, i: (0, 0)),
        ],
        out_specs=[
            pl.BlockSpec((1, bn, H), lambda b, i: (b, i, 0)),
            pl.BlockSpec((1, bn, H), lambda b, i: (b, i, 0)),
            pl.BlockSpec((1, bn, H), lambda b, i: (b, i, 0)),
        ],
        out_shape=[shp, shp, shp],
    )(acc1, v1, h, deg0, deg1, wzr, bz, br)


def _tc_out(acc_a, acc_b, v2, px, z, h, deg0, deg1, whx, whh, bh, B, N, H, bn):
    def body(acca_ref, accb_ref, v2_ref, px_ref, z_ref, h_ref, d0, d1,
             whx_ref, whh_ref, bh_ref, out_ref):
        dinv = _dinv_of(d0, d1)
        prh = (acca_ref[0] + accb_ref[0] + v2_ref[0]) * dinv
        ht = jnp.tanh(jnp.dot(px_ref[0], whx_ref[...], preferred_element_type=_F32)
                      + jnp.dot(prh, whh_ref[...], preferred_element_type=_F32)
                      + bh_ref[...])
        z = z_ref[0]
        out_ref[0] = (1.0 - z) * h_ref[0] + z * ht

    return pl.pallas_call(
        body,
        grid=(B, N // bn),
        in_specs=[
            pl.BlockSpec((1, bn, H), lambda b, i: (b, i, 0)),
            pl.BlockSpec((1, bn, H), lambda b, i: (b, i, 0)),
            pl.BlockSpec((1, bn, H), lambda b, i: (b, i, 0)),
            pl.BlockSpec((1, bn, H), lambda b, i: (b, i, 0)),
            pl.BlockSpec((1, bn, H), lambda b, i: (b, i, 0)),
            pl.BlockSpec((1, bn, H), lambda b, i: (b, i, 0)),
            pl.BlockSpec((bn, 1), lambda b, i: (i, 0)),
            pl.BlockSpec((bn, 1), lambda b, i: (i, 0)),
            pl.BlockSpec((H, H), lambda b, i: (0, 0)),
            pl.BlockSpec((H, H), lambda b, i: (0, 0)),
            pl.BlockSpec((1, H), lambda b, i: (0, 0)),
        ],
        out_specs=pl.BlockSpec((1, bn, H), lambda b, i: (b, i, 0)),
        out_shape=jax.ShapeDtypeStruct((B, N, H), _F32),
    )(acc_a, acc_b, v2, px, z, h, deg0, deg1, whx, whh, bh)


def kernel(x, h, edge_index, W_z, b_z, W_r, b_r, W_h, b_h):
    B, N, F = x.shape
    H = h.shape[2]
    E = edge_index.shape[1]
    assert F == H and E % (_W * _NCORE * _NSUB) == 0
    Npad = -(-N // 2048) * 2048

    nw_deg = E // _W // (_NCORE * _NSUB)
    nb = E // _W // _NSUB // _BW
    src_p = edge_index[0].astype(jnp.int32).reshape(_NSUB, nb, _BW, _W)
    dst_p = edge_index[1].astype(jnp.int32).reshape(_NSUB, nb, _BW, _W)
    dst_d = edge_index[1].astype(jnp.int32).reshape(_NCORE * _NSUB, nw_deg, _W)

    def src_off(nchunks):
        off = (jnp.arange(nchunks, dtype=jnp.int32) * N)
        return src_p[None] + off.reshape(nchunks, 1, 1, 1, 1)

    degp = _make_deg_kernel(Npad, E)(dst_d)
    deg0 = degp[:N].reshape(N, 1)
    deg1 = degp[Npad:Npad + N].reshape(N, 1)

    bn = 2000
    wzr = jnp.concatenate([W_z, W_r], axis=1)
    whx = W_h[:F]
    whh = W_h[F:]
    bz = b_z.reshape(1, H)
    br = b_r.reshape(1, H)
    bh = b_h.reshape(1, H)

    v1 = _tc_scale(x, h, deg0, deg1, B, N, H, bn)
    acc1 = _make_prop_kernel(N, Npad, E, H, 2 * B)(
        v1.reshape(2 * B * N, H), src_off(2 * B), dst_p)
    acc1r = acc1.reshape(B, 2, Npad, H)

    prop_es = _make_prop_kernel(N, Npad, E, H, 1, True)
    src_es = src_p[None]
    outs = []
    for b in range(B):
        v2b, zb, pxb = _tc_gate(acc1r[b:b + 1], v1[b:b + 1], h[b:b + 1],
                                deg0, deg1, wzr, bz, br, 1, N, H, bn)
        pab = prop_es(v2b.reshape(N, H), src_es, dst_p)
        outs.append(_tc_out(pab[0], pab[1], v2b, pxb, zb, h[b:b + 1],
                            deg0, deg1, whx, whh, bh, 1, N, H, bn))
    return jnp.concatenate(outs, axis=0)


# speedup vs baseline: 1.1040x; 1.1040x over previous
import functools

import jax
import jax.numpy as jnp
from jax import lax
from jax.experimental import pallas as pl
from jax.experimental.pallas import tpu as pltpu
from jax.experimental.pallas import tpu_sc as plsc

_W = 80
_NSUB = 16
_NCORE = 2
_F32 = jnp.float32


@functools.lru_cache(maxsize=None)
def _make_deg_kernel(Npad, E):
    nwin_w = E // _W // (_NCORE * _NSUB)
    r0 = Npad // _NSUB
    mesh = plsc.VectorSubcoreMesh(core_axis_name="c", subcore_axis_name="s")

    @functools.partial(
        pl.kernel,
        out_type=jax.ShapeDtypeStruct((_NCORE * Npad,), _F32),
        mesh=mesh,
        scratch_types=[
            pltpu.VMEM_SHARED((Npad,), _F32),
            pltpu.VMEM((nwin_w, _W), jnp.int32),
            pltpu.VMEM((_W,), _F32),
            pltpu.VMEM((r0,), _F32),
        ],
    )
    def deg_kernel(dst3d, out, acc, idxs, ones, zbuf):
        c = lax.axis_index("c")
        s = lax.axis_index("s")

        def fill_ones(i, carry):
            ones[pl.ds(i * 16, 16)] = jnp.ones((16,), _F32)
            return carry

        lax.fori_loop(0, _W // 16, fill_ones, 0)

        def fill_zeros(i, carry):
            zbuf[pl.ds(i * 16, 16)] = jnp.zeros((16,), _F32)
            return carry

        lax.fori_loop(0, r0 // 16, fill_zeros, 0)

        pltpu.sync_copy(zbuf, acc.at[pl.ds(s * r0, r0)])
        plsc.subcore_barrier()

        wid = c * _NSUB + s
        pltpu.sync_copy(dst3d.at[wid], idxs)

        def win(w, carry):
            pltpu.sync_copy(ones, acc.at[idxs.at[w]], add=True)
            return carry

        lax.fori_loop(0, nwin_w, win, 0)
        plsc.subcore_barrier()

        pltpu.sync_copy(acc.at[pl.ds(s * r0, r0)], zbuf)
        pltpu.sync_copy(zbuf, out.at[pl.ds(c * Npad + s * r0, r0)])

    return deg_kernel


_BW = 10


@functools.lru_cache(maxsize=None)
def _make_prop_kernel(N, Npad, E, H, nchunks):
    cpc = nchunks // _NCORE
    nwin = E // _W // _NSUB
    nb = nwin // _BW
    rows_s = Npad // _NSUB
    mesh = plsc.VectorSubcoreMesh(core_axis_name="c", subcore_axis_name="s")

    nring = 4
    nwtot = nb * _BW
    @functools.partial(
        pl.kernel,
        out_type=jax.ShapeDtypeStruct((nchunks, Npad, H), _F32),
        mesh=mesh,
        scratch_types=(
            [pltpu.VMEM_SHARED((Npad, H), _F32)]
            + [pltpu.VMEM((_BW, _W), jnp.int32)] * 4
            + [pltpu.VMEM((_W, H), _F32)] * nring
            + [pltpu.SemaphoreType.DMA] * (2 * nring + 4)
        ),
    )
    def prop_kernel(vflat, src5d, dst4d, out, acc, *rest):
        sidxp = (rest[0], rest[2])
        didxp = (rest[1], rest[3])
        bufs = rest[4:4 + nring]
        gsems = rest[4 + nring:4 + 2 * nring]
        ssems = rest[4 + 2 * nring:4 + 3 * nring]
        isems = rest[4 + 3 * nring:]
        c = lax.axis_index("c")
        s = lax.axis_index("s")

        for ci in range(cpc):
            chunk = c * cpc + ci

            def fill_zeros(i, carry):
                bufs[0][i // (H // 16), pl.ds((i % (H // 16)) * 16, 16)] = \
                    jnp.zeros((16,), _F32)
                return carry

            lax.fori_loop(0, _W * (H // 16), fill_zeros, 0)
            for kk in range(rows_s // _W):
                pltpu.sync_copy(bufs[0], acc.at[pl.ds(s * rows_s + kk * _W, _W)])
            plsc.subcore_barrier()

            icp = [None, None, None, None]

            def idx_load(wb):
                p = wb % 2
                icp[2 * p] = pltpu.async_copy(
                    src5d.at[chunk, s, wb], sidxp[p], isems[2 * p])
                icp[2 * p + 1] = pltpu.async_copy(
                    dst4d.at[s, wb], didxp[p], isems[2 * p + 1])

            idx_load(0)
            if nb > 1:
                idx_load(1)

            pf = nring - 1
            gcp = [None] * nring
            scp = [None] * nring

            def issue_gather(w):
                j = w % nring
                wb, row = w // _BW, w % _BW
                if row == 0:
                    icp[2 * (wb % 2)].wait()
                gcp[j] = pltpu.async_copy(
                    vflat.at[sidxp[wb % 2].at[row]], bufs[j], gsems[j])

            for w in range(min(pf, nwtot)):
                issue_gather(w)
            for w in range(nwtot):
                b = w % nring
                wb, row = w // _BW, w % _BW
                if row == 0:
                    icp[2 * (wb % 2) + 1].wait()
                nxt = w + pf
                if nxt < nwtot:
                    j = nxt % nring
                    if scp[j] is not None:
                        scp[j].wait()
                    issue_gather(nxt)
                gcp[b].wait()
                scp[b] = pltpu.async_copy(
                    bufs[b], acc.at[didxp[wb % 2].at[row]], ssems[b], add=True)
                if row == nring and 1 <= wb < nb - 1:
                    idx_load(wb + 1)
            for j in range(nring):
                if scp[j] is not None:
                    scp[j].wait()
            plsc.subcore_barrier()

            for kk in range(rows_s // _W):
                ro = s * rows_s + kk * _W
                pltpu.sync_copy(acc.at[pl.ds(ro, _W)],
                                out.at[chunk, pl.ds(ro, _W)])
            if ci + 1 < cpc:
                plsc.subcore_barrier()

    return prop_kernel


def _dinv_of(d0, d1):
    return lax.rsqrt(d0[...] + d1[...] + 1.0)


def _tc_scale(x, h, deg0, deg1, B, N, H, bn):
    def body(x_ref, h_ref, d0, d1, v_ref):
        dinv = _dinv_of(d0, d1)
        v_ref[0, 0] = x_ref[0] * dinv
        v_ref[0, 1] = h_ref[0] * dinv

    return pl.pallas_call(
        body,
        grid=(B, N // bn),
        in_specs=[
            pl.BlockSpec((1, bn, H), lambda b, i: (b, i, 0)),
            pl.BlockSpec((1, bn, H), lambda b, i: (b, i, 0)),
            pl.BlockSpec((bn, 1), lambda b, i: (i, 0)),
            pl.BlockSpec((bn, 1), lambda b, i: (i, 0)),
        ],
        out_specs=pl.BlockSpec((1, 2, bn, H), lambda b, i: (b, 0, i, 0)),
        out_shape=jax.ShapeDtypeStruct((B, 2, N, H), _F32),
    )(x, h, deg0, deg1)


def _tc_gate(acc1, v1, h, deg0, deg1, wzr, bz, br, B, N, H, bn):
    def body(acc_ref, v1_ref, h_ref, d0, d1, wzr_ref, bz_ref, br_ref,
             v2_ref, z_ref, px_ref):
        dinv = _dinv_of(d0, d1)
        px = (acc_ref[0, 0] + v1_ref[0, 0]) * dinv
        ph = (acc_ref[0, 1] + v1_ref[0, 1]) * dinv
        zr_pre = (jnp.dot(px, wzr_ref[0:H, :], preferred_element_type=_F32)
                  + jnp.dot(ph, wzr_ref[H:2 * H, :], preferred_element_type=_F32))
        z = jax.nn.sigmoid(zr_pre[:, 0:H] + bz_ref[...])
        r = jax.nn.sigmoid(zr_pre[:, H:2 * H] + br_ref[...])
        v2_ref[0] = (r * h_ref[0]) * dinv
        z_ref[0] = z
        px_ref[0] = px

    shp = jax.ShapeDtypeStruct((B, N, H), _F32)
    return pl.pallas_call(
        body,
        grid=(B, N // bn),
        in_specs=[
            pl.BlockSpec((1, 2, bn, H), lambda b, i: (b, 0, i, 0)),
            pl.BlockSpec((1, 2, bn, H), lambda b, i: (b, 0, i, 0)),
            pl.BlockSpec((1, bn, H), lambda b, i: (b, i, 0)),
            pl.BlockSpec((bn, 1), lambda b, i: (i, 0)),
            pl.BlockSpec((bn, 1), lambda b, i: (i, 0)),
            pl.BlockSpec((2 * H, 2 * H), lambda b, i: (0, 0)),
            pl.BlockSpec((1, H), lambda b, i: (0, 0)),
            pl.BlockSpec((1, H), lambda b, i: (0, 0)),
        ],
        out_specs=[
            pl.BlockSpec((1, bn, H), lambda b, i: (b, i, 0)),
            pl.BlockSpec((1, bn, H), lambda b, i: (b, i, 0)),
            pl.BlockSpec((1, bn, H), lambda b, i: (b, i, 0)),
        ],
        out_shape=[shp, shp, shp],
    )(acc1, v1, h, deg0, deg1, wzr, bz, br)


def _tc_out(acc2, v2, px, z, h, deg0, deg1, whx, whh, bh, B, N, H, bn):
    def body(acc_ref, v2_ref, px_ref, z_ref, h_ref, d0, d1,
             whx_ref, whh_ref, bh_ref, out_ref):
        dinv = _dinv_of(d0, d1)
        prh = (acc_ref[0] + v2_ref[0]) * dinv
        ht = jnp.tanh(jnp.dot(px_ref[0], whx_ref[...], preferred_element_type=_F32)
                      + jnp.dot(prh, whh_ref[...], preferred_element_type=_F32)
                      + bh_ref[...])
        z = z_ref[0]
        out_ref[0] = (1.0 - z) * h_ref[0] + z * ht

    return pl.pallas_call(
        body,
        grid=(B, N // bn),
        in_specs=[
            pl.BlockSpec((1, bn, H), lambda b, i: (b, i, 0)),
            pl.BlockSpec((1, bn, H), lambda b, i: (b, i, 0)),
            pl.BlockSpec((1, bn, H), lambda b, i: (b, i, 0)),
            pl.BlockSpec((1, bn, H), lambda b, i: (b, i, 0)),
            pl.BlockSpec((1, bn, H), lambda b, i: (b, i, 0)),
            pl.BlockSpec((bn, 1), lambda b, i: (i, 0)),
            pl.BlockSpec((bn, 1), lambda b, i: (i, 0)),
            pl.BlockSpec((H, H), lambda b, i: (0, 0)),
            pl.BlockSpec((H, H), lambda b, i: (0, 0)),
            pl.BlockSpec((1, H), lambda b, i: (0, 0)),
        ],
        out_specs=pl.BlockSpec((1, bn, H), lambda b, i: (b, i, 0)),
        out_shape=jax.ShapeDtypeStruct((B, N, H), _F32),
    )(acc2, v2, px, z, h, deg0, deg1, whx, whh, bh)


def kernel(x, h, edge_index, W_z, b_z, W_r, b_r, W_h, b_h):
    B, N, F = x.shape
    H = h.shape[2]
    E = edge_index.shape[1]
    assert F == H and E % (_W * _NCORE * _NSUB) == 0
    Npad = -(-N // 2048) * 2048

    nw_deg = E // _W // (_NCORE * _NSUB)
    nb = E // _W // _NSUB // _BW
    src_p = edge_index[0].astype(jnp.int32).reshape(_NSUB, nb, _BW, _W)
    dst_p = edge_index[1].astype(jnp.int32).reshape(_NSUB, nb, _BW, _W)
    dst_d = edge_index[1].astype(jnp.int32).reshape(_NCORE * _NSUB, nw_deg, _W)

    def src_off(nchunks):
        off = (jnp.arange(nchunks, dtype=jnp.int32) * N)
        return src_p[None] + off.reshape(nchunks, 1, 1, 1, 1)

    degp = _make_deg_kernel(Npad, E)(dst_d)
    deg0 = degp[:N].reshape(N, 1)
    deg1 = degp[Npad:Npad + N].reshape(N, 1)

    bn = 2000
    wzr = jnp.concatenate([W_z, W_r], axis=1)
    whx = W_h[:F]
    whh = W_h[F:]
    bz = b_z.reshape(1, H)
    br = b_r.reshape(1, H)
    bh = b_h.reshape(1, H)

    v1 = _tc_scale(x, h, deg0, deg1, B, N, H, bn)
    acc1 = _make_prop_kernel(N, Npad, E, H, 2 * B)(
        v1.reshape(2 * B * N, H), src_off(2 * B), dst_p)
    acc1r = acc1.reshape(B, 2, Npad, H)

    v2, z, px = _tc_gate(acc1r, v1, h, deg0, deg1, wzr, bz, br, B, N, H, bn)

    acc2 = _make_prop_kernel(N, Npad, E, H, B)(
        v2.reshape(B * N, H), src_off(B), dst_p)
    return _tc_out(acc2, v2, px, z, h, deg0, deg1, whx, whh, bh, B, N, H, bn)
